# baseline (speedup 1.0000x reference)
import functools

import jax
import jax.numpy as jnp
from jax import lax
from jax.experimental import pallas as pl
from jax.experimental.pallas import tpu as pltpu
from jax.experimental.pallas import tpu_sc as plsc

NUM_WORKERS = 32
CHUNK = 128
NBUF = 6
SKEW = 2


def kernel(class_ids, table):
    B, T = class_ids.shape
    V, D = table.shape
    total = B * T
    per_w = total // NUM_WORKERS
    n_chunks = per_w // CHUNK
    ids2d = class_ids.reshape(total // CHUNK, CHUNK).astype(jnp.int32)

    mesh = plsc.VectorSubcoreMesh(core_axis_name="c", subcore_axis_name="s")

    @functools.partial(
        pl.kernel,
        out_type=jax.ShapeDtypeStruct((total, D), jnp.float32),
        mesh=mesh,
        scratch_types=[
            pltpu.VMEM((n_chunks, CHUNK), jnp.int32),
            [pltpu.VMEM((CHUNK, D), jnp.float32) for _ in range(NBUF)],
            pltpu.VMEM_SHARED((V, D), jnp.float32),
            pltpu.SemaphoreType.DMA((NBUF,)),
            pltpu.SemaphoreType.DMA((NBUF,)),
        ],
    )
    def sc_gather(ids_hbm, table_hbm, out_hbm, idx_v, bufs, table_sh, gsem, ssem):
        wid = lax.axis_index("s") * 2 + lax.axis_index("c")

        @pl.when(lax.axis_index("s") == 0)
        def _():
            pltpu.sync_copy(table_hbm, table_sh)

        plsc.subcore_barrier()

        pltpu.sync_copy(ids_hbm.at[pl.ds(wid * n_chunks, n_chunks)], idx_v)
        base = wid * per_w

        def gather(j, b):
            return pltpu.make_async_copy(
                table_sh.at[idx_v.at[j]], bufs[b], gsem.at[b])

        def scatter(j, b):
            return pltpu.make_async_copy(
                bufs[b], out_hbm.at[pl.ds(base + j * CHUNK, CHUNK)], ssem.at[b])

        for b in range(NBUF):
            gather(b, b).start()
        for j in range(n_chunks):
            b = j % NBUF
            gather(j, b).wait()
            scatter(j, b).start()
            jp = j - SKEW
            if jp >= 0:
                scatter(jp, jp % NBUF).wait()
                if jp + NBUF < n_chunks:
                    gather(jp + NBUF, jp % NBUF).start()
        for j in range(n_chunks - SKEW, n_chunks):
            scatter(j, j % NBUF).wait()

    out = sc_gather(ids2d, table)
    return out.reshape(B, T, D)


# speedup vs baseline: 1.0362x; 1.0362x over previous
import functools

import jax
import jax.numpy as jnp
from jax import lax
from jax.experimental import pallas as pl
from jax.experimental.pallas import tpu as pltpu
from jax.experimental.pallas import tpu_sc as plsc

NUM_WORKERS = 32
CHUNK = 128
NBUF = 4
SKEW = 1


def kernel(class_ids, table):
    B, T = class_ids.shape
    V, D = table.shape
    total = B * T
    per_w = total // NUM_WORKERS
    n_chunks = per_w // CHUNK
    ids2d = class_ids.reshape(total // CHUNK, CHUNK).astype(jnp.int32)

    mesh = plsc.VectorSubcoreMesh(core_axis_name="c", subcore_axis_name="s")

    @functools.partial(
        pl.kernel,
        out_type=jax.ShapeDtypeStruct((total, D), jnp.float32),
        mesh=mesh,
        scratch_types=[
            pltpu.VMEM((n_chunks, CHUNK), jnp.int32),
            [pltpu.VMEM((CHUNK, D), jnp.float32) for _ in range(NBUF)],
            pltpu.VMEM_SHARED((V, D), jnp.float32),
            pltpu.SemaphoreType.DMA((NBUF,)),
            pltpu.SemaphoreType.DMA((NBUF,)),
        ],
    )
    def sc_gather(ids_hbm, table_hbm, out_hbm, idx_v, bufs, table_sh, gsem, ssem):
        wid = lax.axis_index("s") * 2 + lax.axis_index("c")

        @pl.when(lax.axis_index("s") == 0)
        def _():
            pltpu.sync_copy(table_hbm, table_sh)

        plsc.subcore_barrier()

        pltpu.sync_copy(ids_hbm.at[pl.ds(wid * n_chunks, n_chunks)], idx_v)
        base = wid * per_w

        def gather(j, b):
            return pltpu.make_async_copy(
                table_sh.at[idx_v.at[j]], bufs[b], gsem.at[b])

        def scatter(j, b):
            return pltpu.make_async_copy(
                bufs[b], out_hbm.at[pl.ds(base + j * CHUNK, CHUNK)], ssem.at[b])

        for b in range(NBUF):
            gather(b, b).start()

        @pl.loop(0, n_chunks // NBUF)
        def _(g):
            for b in range(NBUF):
                j = g * NBUF + b
                gather(j, b).wait()
                scatter(j, b).start()
                jp = j - SKEW
                bp = (b - SKEW) % NBUF

                @pl.when(jp >= 0)
                def _():
                    scatter(jp, bp).wait()

                    @pl.when(jp + NBUF < n_chunks)
                    def _():
                        gather(jp + NBUF, bp).start()

        for j in range(n_chunks - SKEW, n_chunks):
            scatter(j, j % NBUF).wait()

    out = sc_gather(ids2d, table)
    return out.reshape(B, T, D)


# speedup vs baseline: 1.0520x; 1.0152x over previous
import functools

import jax
import jax.numpy as jnp
from jax import lax
from jax.experimental import pallas as pl
from jax.experimental.pallas import tpu as pltpu
from jax.experimental.pallas import tpu_sc as plsc

NUM_WORKERS = 32
CHUNK = 128
NBUF = 4
SKEW = 1


def kernel(class_ids, table):
    B, T = class_ids.shape
    V, D = table.shape
    total = B * T
    per_w = total // NUM_WORKERS
    n_chunks = per_w // CHUNK
    ids2d = class_ids.reshape(total // CHUNK, CHUNK).astype(jnp.int32)

    mesh = plsc.VectorSubcoreMesh(core_axis_name="c", subcore_axis_name="s")

    @functools.partial(
        pl.kernel,
        out_type=jax.ShapeDtypeStruct((total, D), jnp.float32),
        mesh=mesh,
        scratch_types=[
            pltpu.VMEM((n_chunks, CHUNK), jnp.int32),
            [pltpu.VMEM((CHUNK, D), jnp.float32) for _ in range(NBUF)],
            pltpu.VMEM_SHARED((V, D), jnp.float32),
            pltpu.SemaphoreType.DMA((NBUF,)),
            pltpu.SemaphoreType.DMA((NBUF,)),
            pltpu.SemaphoreType.DMA,
        ],
    )
    def sc_gather(ids_hbm, table_hbm, out_hbm, idx_v, bufs, table_sh, gsem, ssem,
                  tsem):
        wid = lax.axis_index("s") * 2 + lax.axis_index("c")

        tstage = pltpu.make_async_copy(table_hbm, table_sh, tsem)

        @pl.when(lax.axis_index("s") == 0)
        def _():
            tstage.start()

        pltpu.sync_copy(ids_hbm.at[pl.ds(wid * n_chunks, n_chunks)], idx_v)

        @pl.when(lax.axis_index("s") == 0)
        def _():
            tstage.wait()

        plsc.subcore_barrier()
        base = wid * per_w

        def gather(j, b):
            return pltpu.make_async_copy(
                table_sh.at[idx_v.at[j]], bufs[b], gsem.at[b])

        def scatter(j, b):
            return pltpu.make_async_copy(
                bufs[b], out_hbm.at[pl.ds(base + j * CHUNK, CHUNK)], ssem.at[b])

        for b in range(NBUF):
            gather(b, b).start()

        @pl.loop(0, n_chunks // NBUF)
        def _(g):
            for b in range(NBUF):
                j = g * NBUF + b
                gather(j, b).wait()
                scatter(j, b).start()
                jp = j - SKEW
                bp = (b - SKEW) % NBUF

                @pl.when(jp >= 0)
                def _():
                    scatter(jp, bp).wait()

                    @pl.when(jp + NBUF < n_chunks)
                    def _():
                        gather(jp + NBUF, bp).start()

        for j in range(n_chunks - SKEW, n_chunks):
            scatter(j, j % NBUF).wait()

    out = sc_gather(ids2d, table)
    return out.reshape(B, T, D)
